# baseline (device time: 77943 ns/iter reference)
import jax
import jax.numpy as jnp
from jax import lax
from jax.experimental import pallas as pl
from jax.experimental.pallas import tpu as pltpu

LOG2E = 1.4426950408889634


def kernel(Q, K, V):
    b, s, h, d = Q.shape
    scale = d ** -0.5

    def body(q_ref, k_ref, v_ref, o_ref,
             krecv, vrecv, acc_ref,
             zsend, zrecv, fsend, xrecv, yrecv):
        ph = pl.program_id(0)
        bo = pl.program_id(1)
        hi = pl.program_id(2)
        my_x = lax.axis_index("x")
        my_y = lax.axis_index("y")
        my_z = lax.axis_index("z")
        partner = (my_x, my_y, 1 - my_z)
        xnbr = (1 - my_x, my_y, my_z)
        ynbr = (my_x, 1 - my_y, my_z)

        own_b = 2 * my_x + my_y
        xn_b = 2 * (1 - my_x) + my_y
        yn_b = 2 * my_x + (1 - my_y)
        diag_b = 2 * (1 - my_x) + (1 - my_y)

        def z_copy(src, dst, slot, i):
            return pltpu.make_async_remote_copy(
                src_ref=src.at[slot], dst_ref=dst.at[slot],
                send_sem=zsend.at[i], recv_sem=zrecv.at[i],
                device_id=partner, device_id_type=pl.DeviceIdType.MESH,
            )

        def fwd_copy(buf, i, nbr, rsem):
            return pltpu.make_async_remote_copy(
                src_ref=buf.at[own_b], dst_ref=buf.at[own_b],
                send_sem=fsend.at[i], recv_sem=rsem,
                device_id=nbr, device_id_type=pl.DeviceIdType.MESH,
            )

        zk_own = lambda: z_copy(k_ref, krecv, own_b, 0)
        zv_own = lambda: z_copy(v_ref, vrecv, own_b, 1)
        zk_diag = lambda: z_copy(k_ref, krecv, diag_b, 2)
        zv_diag = lambda: z_copy(v_ref, vrecv, diag_b, 3)
        xk = lambda: fwd_copy(krecv, 0, xnbr, xrecv.at[0])
        xv = lambda: fwd_copy(vrecv, 1, xnbr, xrecv.at[1])
        yk = lambda: fwd_copy(krecv, 2, ynbr, yrecv.at[0])
        yv = lambda: fwd_copy(vrecv, 3, ynbr, yrecv.at[1])

        @pl.when((ph == 0) & (bo == 0) & (hi == 0) & (my_x > 9))
        def _():
            barrier_sem = pltpu.get_barrier_semaphore()
            for nbr in (partner, xnbr, ynbr):
                pl.semaphore_signal(
                    barrier_sem, inc=1,
                    device_id=nbr, device_id_type=pl.DeviceIdType.MESH,
                )
            pl.semaphore_wait(barrier_sem, 3)
            zk_own().start()
            zv_own().start()
            zk_diag().start()
            zv_diag().start()

        @pl.when((ph == 0) & (bo == 3) & (hi == 0) & (my_x > 9))
        def _():
            zk_own().wait_recv()
            xk().start()
            yk().start()

        @pl.when((ph == 1) & (bo == 0) & (hi == 0) & (my_x > 9))
        def _():
            zv_own().wait_recv()
            xv().start()
            yv().start()

        @pl.when((ph == 1) & (bo == 1) & (hi == 0) & (my_x > 9))
        def _():
            xk().wait_recv()
            xv().wait_recv()

        @pl.when((ph == 1) & (bo == 2) & (hi == 0) & (my_x > 9))
        def _():
            yk().wait_recv()
            yv().wait_recv()

        @pl.when((ph == 1) & (bo == 3) & (hi == 0) & (my_x > 9))
        def _():
            zk_diag().wait_recv()
            zv_diag().wait_recv()

        b_act = jnp.where(
            bo == 0, own_b,
            jnp.where(bo == 1, xn_b, jnp.where(bo == 2, yn_b, diag_b)),
        )

        q = q_ref[b_act, hi]

        @pl.when(ph == 0)
        def _():
            sl = lax.dot_general(
                q, k_ref[b_act, hi], (((1,), (0,)), ((), ())),
                preferred_element_type=jnp.float32,
            )
            el = jnp.exp2(sl).astype(jnp.bfloat16)
            acc_ref[b_act, hi] = lax.dot_general(
                el, v_ref[b_act, hi], (((1,), (0,)), ((), ())),
                preferred_element_type=jnp.float32,
            )

        @pl.when(ph == 1)
        def _():
            sr = lax.dot_general(
                q, k_ref[b_act, hi], (((1,), (0,)), ((), ())),
                preferred_element_type=jnp.float32,
            )
            er = jnp.exp2(sr).astype(jnp.bfloat16)
            acc = acc_ref[b_act, hi] + lax.dot_general(
                er, v_ref[b_act, hi], (((1,), (0,)), ((), ())),
                preferred_element_type=jnp.float32,
            )
            o_ref[b_act, hi] = (acc[:, :d] / acc[:, d:]).astype(jnp.bfloat16)

        @pl.when((ph == 1) & (bo == 3) & (hi == h - 1) & (my_x > 9))
        def _():
            zk_own().wait_send()
            zv_own().wait_send()
            zk_diag().wait_send()
            zv_diag().wait_send()
            xk().wait_send()
            xv().wait_send()
            yk().wait_send()
            yv().wait_send()

    qt = (jnp.transpose(Q, (0, 2, 1, 3)) * (scale * LOG2E)).astype(jnp.bfloat16)
    kt = jnp.transpose(K, (0, 2, 3, 1)).astype(jnp.bfloat16)
    vt = jnp.transpose(V, (0, 2, 1, 3)).astype(jnp.bfloat16)
    vt = jnp.concatenate(
        [vt, jnp.ones((b, h, s, 1), jnp.bfloat16)], axis=-1
    )

    out_t = pl.pallas_call(
        body,
        grid=(2, b, h),
        out_shape=jax.ShapeDtypeStruct((b, h, s, d), jnp.bfloat16),
        in_specs=[pl.BlockSpec(memory_space=pltpu.VMEM)] * 3,
        out_specs=pl.BlockSpec(memory_space=pltpu.VMEM),
        scratch_shapes=[
            pltpu.VMEM((b, h, d, s), jnp.bfloat16),
            pltpu.VMEM((b, h, s, d + 1), jnp.bfloat16),
            pltpu.VMEM((b, h, s, d + 1), jnp.float32),
            pltpu.SemaphoreType.DMA((4,)),
            pltpu.SemaphoreType.DMA((4,)),
            pltpu.SemaphoreType.DMA((4,)),
            pltpu.SemaphoreType.DMA((2,)),
            pltpu.SemaphoreType.DMA((2,)),
        ],
        compiler_params=pltpu.CompilerParams(collective_id=0),
    )(qt, kt, vt)
    return jnp.transpose(out_t, (0, 2, 1, 3)).astype(jnp.float32)


# device time: 30637 ns/iter; 2.5441x vs baseline; 2.5441x over previous
import jax
import jax.numpy as jnp
from jax import lax
from jax.experimental import pallas as pl
from jax.experimental.pallas import tpu as pltpu

LOG2E = 1.4426950408889634


def kernel(Q, K, V):
    b, s, h, d = Q.shape
    scale = d ** -0.5

    def body(q_ref, k_ref, v_ref, o_ref, acc_ref):
        ph = pl.program_id(0)
        bi = pl.program_id(1)

        q3 = q_ref[bi]
        k3 = k_ref[bi]
        v3 = v_ref[bi]
        s3 = lax.dot_general(
            q3, k3, (((2,), (1,)), ((0,), (0,))),
            preferred_element_type=jnp.float32,
        )
        e3 = jnp.exp2(s3).astype(jnp.bfloat16)
        p3 = lax.dot_general(
            e3, v3, (((2,), (1,)), ((0,), (0,))),
            preferred_element_type=jnp.float32,
        )

        @pl.when(ph == 0)
        def _():
            acc_ref[bi] = p3

        @pl.when(ph == 1)
        def _():
            a = acc_ref[bi] + p3
            o_ref[bi] = (a[:, :, :d] / a[:, :, d:]).astype(jnp.bfloat16)

    qt = (jnp.transpose(Q, (0, 2, 1, 3)) * (scale * LOG2E)).astype(jnp.bfloat16)
    kt = jnp.transpose(K, (0, 2, 3, 1)).astype(jnp.bfloat16)
    vt = jnp.transpose(V, (0, 2, 1, 3)).astype(jnp.bfloat16)
    vt = jnp.concatenate(
        [vt, jnp.ones((b, h, s, 1), jnp.bfloat16)], axis=-1
    )

    out_t = pl.pallas_call(
        body,
        grid=(2, b),
        out_shape=jax.ShapeDtypeStruct((b, h, s, d), jnp.bfloat16),
        in_specs=[pl.BlockSpec(memory_space=pltpu.VMEM)] * 3,
        out_specs=pl.BlockSpec(memory_space=pltpu.VMEM),
        scratch_shapes=[
            pltpu.VMEM((b, h, s, d + 1), jnp.float32),
        ],
    )(qt, kt, vt)
    return jnp.transpose(out_t, (0, 2, 1, 3)).astype(jnp.float32)
